# Initial kernel scaffold; baseline (speedup 1.0000x reference)
import jax
import jax.numpy as jnp
from jax.experimental import pallas as pl


def kernel(x, edge_index, W_pre, b_pre, W1_0, b1_0, W2_0, b2_0, mem_0, W1_1, b1_1, W2_1, b2_1, mem_1, W_lin, b_lin):
    raise NotImplementedError("write your pallas kernel here")


# speedup vs baseline: 6.4049x; 6.4049x over previous
import functools

import jax
import jax.numpy as jnp
from jax import lax
from jax.experimental import pallas as pl
from jax.experimental.pallas import tpu as pltpu
from jax.experimental.pallas import tpu_sc as plsc

N = 10000
E = 320000
NFEAT = 128
NHID = 128
NCLASS = 40
NMEM = 8

NC = 2
NS = 16
NW = NC * NS
EW = E // NW
K = 80
CHUNKS = EW // K
RPT = 624
RPT_LAST = N - (NS - 1) * RPT

BLK = 400
GRID = N // BLK


def _seg_sum_body(h_hbm, src_hbm, dst_hbm, zeros_hbm, out_hbm,
                  src_v, dst_v, rows_v, acc):
    cid = lax.axis_index("c")
    sid = lax.axis_index("s")
    wid = cid * NS + sid

    pltpu.sync_copy(src_hbm.at[wid], src_v)
    pltpu.sync_copy(dst_hbm.at[wid], dst_v)

    @pl.when(sid < NS - 1)
    def _():
        pltpu.sync_copy(zeros_hbm.at[pl.ds(sid * RPT, RPT)],
                        acc.at[pl.ds(sid * RPT, RPT)])

    @pl.when(sid == NS - 1)
    def _():
        pltpu.sync_copy(zeros_hbm.at[pl.ds((NS - 1) * RPT, RPT_LAST)],
                        acc.at[pl.ds((NS - 1) * RPT, RPT_LAST)])

    plsc.subcore_barrier()

    def step(j, carry):
        pltpu.sync_copy(h_hbm.at[src_v.at[j]], rows_v)
        pltpu.sync_copy(rows_v, acc.at[dst_v.at[j]], add=True)
        return carry

    lax.fori_loop(0, CHUNKS, step, 0)
    plsc.subcore_barrier()

    @pl.when(sid < NS - 1)
    def _():
        pltpu.sync_copy(acc.at[pl.ds(sid * RPT, RPT)],
                        out_hbm.at[cid, pl.ds(sid * RPT, RPT)])

    @pl.when(sid == NS - 1)
    def _():
        pltpu.sync_copy(acc.at[pl.ds((NS - 1) * RPT, RPT_LAST)],
                        out_hbm.at[cid, pl.ds((NS - 1) * RPT, RPT_LAST)])


_seg_sum = functools.partial(
    pl.kernel,
    _seg_sum_body,
    out_type=jax.ShapeDtypeStruct((NC, N, NHID), jnp.float32),
    mesh=plsc.VectorSubcoreMesh(core_axis_name="c", subcore_axis_name="s"),
    scratch_types=[
        pltpu.VMEM((CHUNKS, K), jnp.int32),
        pltpu.VMEM((CHUNKS, K), jnp.int32),
        pltpu.VMEM((K, NHID), jnp.float32),
        pltpu.VMEM_SHARED((N, NHID), jnp.float32),
    ],
)()


def _pre_body(x_ref, w_ref, b_ref, o_ref):
    o_ref[...] = x_ref[...] @ w_ref[...] + b_ref[...]


def _layer_body(h_ref, aa_ref, ab_ref, w1x_ref, w1a_ref, b1_ref,
                w2_ref, b2_ref, memt_ref, mem_ref, o_ref):
    h = h_ref[...]
    agg = aa_ref[...] + ab_ref[...]
    t = h @ w1x_ref[...] + agg @ w1a_ref[...] + b1_ref[...]
    t = jnp.maximum(t, 0.0)
    u = t @ w2_ref[...] + b2_ref[...]
    u = jnp.maximum(u, 0.0)
    s = u @ memt_ref[...]
    s = s - jnp.max(s, axis=-1, keepdims=True)
    e = jnp.exp(s)
    p = e / jnp.sum(e, axis=-1, keepdims=True)
    o_ref[...] = p @ mem_ref[...]


def _final_body(h_ref, aa_ref, ab_ref, w1x_ref, w1a_ref, b1_ref,
                w2_ref, b2_ref, memt_ref, mem_ref,
                wla_ref, wlb_ref, bl_ref, o_ref):
    h = h_ref[...]
    agg = aa_ref[...] + ab_ref[...]
    t = h @ w1x_ref[...] + agg @ w1a_ref[...] + b1_ref[...]
    t = jnp.maximum(t, 0.0)
    u = t @ w2_ref[...] + b2_ref[...]
    u = jnp.maximum(u, 0.0)
    s = u @ memt_ref[...]
    s = s - jnp.max(s, axis=-1, keepdims=True)
    e = jnp.exp(s)
    p = e / jnp.sum(e, axis=-1, keepdims=True)
    h2 = p @ mem_ref[...]
    y = h @ wla_ref[...] + h2 @ wlb_ref[...] + bl_ref[...]
    m = jnp.max(y, axis=-1, keepdims=True)
    z = y - m
    o_ref[...] = z - jnp.log(jnp.sum(jnp.exp(z), axis=-1, keepdims=True))


def _row_spec(cols):
    return pl.BlockSpec((BLK, cols), lambda i: (i, 0))


def _full_spec(rows, cols):
    return pl.BlockSpec((rows, cols), lambda i: (0, 0))


def kernel(x, edge_index, W_pre, b_pre, W1_0, b1_0, W2_0, b2_0, mem_0,
           W1_1, b1_1, W2_1, b2_1, mem_1, W_lin, b_lin):
    src = edge_index[0].astype(jnp.int32).reshape(NW, CHUNKS, K)
    dst = edge_index[1].astype(jnp.int32).reshape(NW, CHUNKS, K)
    zeros = jnp.zeros((N, NHID), jnp.float32)

    pre = pl.pallas_call(
        _pre_body,
        grid=(GRID,),
        in_specs=[_row_spec(NFEAT), _full_spec(NFEAT, NHID), _full_spec(1, NHID)],
        out_specs=_row_spec(NHID),
        out_shape=jax.ShapeDtypeStruct((N, NHID), jnp.float32),
    )
    h0 = pre(x, W_pre, b_pre.reshape(1, NHID))

    layer = pl.pallas_call(
        _layer_body,
        grid=(GRID,),
        in_specs=[
            _row_spec(NHID), _row_spec(NHID), _row_spec(NHID),
            _full_spec(NHID, NHID), _full_spec(NHID, NHID), _full_spec(1, NHID),
            _full_spec(NHID, NHID), _full_spec(1, NHID),
            _full_spec(NHID, NMEM), _full_spec(NMEM, NHID),
        ],
        out_specs=_row_spec(NHID),
        out_shape=jax.ShapeDtypeStruct((N, NHID), jnp.float32),
    )

    final = pl.pallas_call(
        _final_body,
        grid=(GRID,),
        in_specs=[
            _row_spec(NHID), _row_spec(NHID), _row_spec(NHID),
            _full_spec(NHID, NHID), _full_spec(NHID, NHID), _full_spec(1, NHID),
            _full_spec(NHID, NHID), _full_spec(1, NHID),
            _full_spec(NHID, NMEM), _full_spec(NMEM, NHID),
            _full_spec(NHID, NCLASS), _full_spec(NHID, NCLASS), _full_spec(1, NCLASS),
        ],
        out_specs=_row_spec(NCLASS),
        out_shape=jax.ShapeDtypeStruct((N, NCLASS), jnp.float32),
    )

    part0 = _seg_sum(h0, src, dst, zeros)
    h1 = layer(h0, part0[0], part0[1],
               W1_0[:NHID], W1_0[NHID:], b1_0.reshape(1, NHID),
               W2_0, b2_0.reshape(1, NHID), mem_0.T, mem_0)

    part1 = _seg_sum(h1, src, dst, zeros)
    out = final(h1, part1[0], part1[1],
                W1_1[:NHID], W1_1[NHID:], b1_1.reshape(1, NHID),
                W2_1, b2_1.reshape(1, NHID), mem_1.T, mem_1,
                W_lin[:NHID], W_lin[NHID:], b_lin.reshape(1, NCLASS))
    return out


# speedup vs baseline: 9.6030x; 1.4993x over previous
import functools

import jax
import jax.numpy as jnp
from jax import lax
from jax.experimental import pallas as pl
from jax.experimental.pallas import tpu as pltpu
from jax.experimental.pallas import tpu_sc as plsc

N = 10000
E = 320000
NFEAT = 128
NHID = 128
NCLASS = 40
NMEM = 8

NC = 2
NS = 16
NW = NC * NS
EW = E // NW
K = 80
CHUNKS = EW // K
NBUF = 2
RPT = 624
RPT_LAST = N - (NS - 1) * RPT

BLK = 400
GRID = N // BLK


def _seg_sum_body(h_hbm, src_hbm, dst_hbm, zeros_hbm, out_hbm,
                  src_v, dst_v, rows0, rows1, acc, *gsems):
    rows = [rows0, rows1]
    cid = lax.axis_index("c")
    sid = lax.axis_index("s")
    wid = cid * NS + sid

    pltpu.sync_copy(src_hbm.at[wid], src_v)
    pltpu.sync_copy(dst_hbm.at[wid], dst_v)

    for b in range(NBUF):
        pltpu.async_copy(h_hbm.at[src_v.at[pl.ds(b * K, K)]], rows[b],
                         gsems[b])

    @pl.when(sid < NS - 1)
    def _():
        pltpu.sync_copy(zeros_hbm.at[pl.ds(sid * RPT, RPT)],
                        acc.at[pl.ds(sid * RPT, RPT)])

    @pl.when(sid == NS - 1)
    def _():
        pltpu.sync_copy(zeros_hbm.at[pl.ds((NS - 1) * RPT, RPT_LAST)],
                        acc.at[pl.ds((NS - 1) * RPT, RPT_LAST)])

    plsc.subcore_barrier()

    def step(jj, carry):
        for b in range(NBUF):
            j = jj * NBUF + b
            pltpu.make_async_copy(h_hbm.at[src_v.at[pl.ds(j * K, K)]],
                                  rows[b], gsems[b]).wait()
            pltpu.sync_copy(rows[b], acc.at[dst_v.at[j]], add=True)

            @pl.when(j + NBUF < CHUNKS)
            def _():
                pltpu.async_copy(
                    h_hbm.at[src_v.at[pl.ds((j + NBUF) * K, K)]], rows[b],
                    gsems[b])
        return carry

    lax.fori_loop(0, CHUNKS // NBUF, step, 0)
    for j in range((CHUNKS // NBUF) * NBUF, CHUNKS):
        b = j % NBUF
        pltpu.make_async_copy(h_hbm.at[src_v.at[pl.ds(j * K, K)]],
                              rows[b], gsems[b]).wait()
        pltpu.sync_copy(rows[b], acc.at[dst_v.at[j]], add=True)
    plsc.subcore_barrier()

    @pl.when(sid < NS - 1)
    def _():
        pltpu.sync_copy(acc.at[pl.ds(sid * RPT, RPT)],
                        out_hbm.at[cid, pl.ds(sid * RPT, RPT)])

    @pl.when(sid == NS - 1)
    def _():
        pltpu.sync_copy(acc.at[pl.ds((NS - 1) * RPT, RPT_LAST)],
                        out_hbm.at[cid, pl.ds((NS - 1) * RPT, RPT_LAST)])


_seg_sum = functools.partial(
    pl.kernel,
    _seg_sum_body,
    out_type=jax.ShapeDtypeStruct((NC, N, NHID), jnp.float32),
    mesh=plsc.VectorSubcoreMesh(core_axis_name="c", subcore_axis_name="s"),
    scratch_types=[
        pltpu.VMEM((EW,), jnp.int32),
        pltpu.VMEM((CHUNKS, K), jnp.int32),
        pltpu.VMEM((K, NHID), jnp.float32),
        pltpu.VMEM((K, NHID), jnp.float32),
        pltpu.VMEM_SHARED((N, NHID), jnp.float32),
    ] + [pltpu.SemaphoreType.DMA] * NBUF,
)()


def _pre_body(x_ref, w_ref, b_ref, o_ref):
    o_ref[...] = x_ref[...] @ w_ref[...] + b_ref[...]


def _layer_body(h_ref, aa_ref, ab_ref, w1x_ref, w1a_ref, b1_ref,
                w2_ref, b2_ref, memt_ref, mem_ref, o_ref):
    h = h_ref[...]
    agg = aa_ref[...] + ab_ref[...]
    t = h @ w1x_ref[...] + agg @ w1a_ref[...] + b1_ref[...]
    t = jnp.maximum(t, 0.0)
    u = t @ w2_ref[...] + b2_ref[...]
    u = jnp.maximum(u, 0.0)
    s = u @ memt_ref[...]
    s = s - jnp.max(s, axis=-1, keepdims=True)
    e = jnp.exp(s)
    p = e / jnp.sum(e, axis=-1, keepdims=True)
    o_ref[...] = p @ mem_ref[...]


def _final_body(h_ref, aa_ref, ab_ref, w1x_ref, w1a_ref, b1_ref,
                w2_ref, b2_ref, memt_ref, mem_ref,
                wla_ref, wlb_ref, bl_ref, o_ref):
    h = h_ref[...]
    agg = aa_ref[...] + ab_ref[...]
    t = h @ w1x_ref[...] + agg @ w1a_ref[...] + b1_ref[...]
    t = jnp.maximum(t, 0.0)
    u = t @ w2_ref[...] + b2_ref[...]
    u = jnp.maximum(u, 0.0)
    s = u @ memt_ref[...]
    s = s - jnp.max(s, axis=-1, keepdims=True)
    e = jnp.exp(s)
    p = e / jnp.sum(e, axis=-1, keepdims=True)
    h2 = p @ mem_ref[...]
    y = h @ wla_ref[...] + h2 @ wlb_ref[...] + bl_ref[...]
    m = jnp.max(y, axis=-1, keepdims=True)
    z = y - m
    o_ref[...] = z - jnp.log(jnp.sum(jnp.exp(z), axis=-1, keepdims=True))


def _row_spec(cols):
    return pl.BlockSpec((BLK, cols), lambda i: (i, 0))


def _full_spec(rows, cols):
    return pl.BlockSpec((rows, cols), lambda i: (0, 0))


def kernel(x, edge_index, W_pre, b_pre, W1_0, b1_0, W2_0, b2_0, mem_0,
           W1_1, b1_1, W2_1, b2_1, mem_1, W_lin, b_lin):
    src = edge_index[0].astype(jnp.int32).reshape(NW, EW)
    dst = edge_index[1].astype(jnp.int32).reshape(NW, CHUNKS, K)
    zeros = jnp.zeros((N, NHID), jnp.float32)

    pre = pl.pallas_call(
        _pre_body,
        grid=(GRID,),
        in_specs=[_row_spec(NFEAT), _full_spec(NFEAT, NHID), _full_spec(1, NHID)],
        out_specs=_row_spec(NHID),
        out_shape=jax.ShapeDtypeStruct((N, NHID), jnp.float32),
    )
    h0 = pre(x, W_pre, b_pre.reshape(1, NHID))

    layer = pl.pallas_call(
        _layer_body,
        grid=(GRID,),
        in_specs=[
            _row_spec(NHID), _row_spec(NHID), _row_spec(NHID),
            _full_spec(NHID, NHID), _full_spec(NHID, NHID), _full_spec(1, NHID),
            _full_spec(NHID, NHID), _full_spec(1, NHID),
            _full_spec(NHID, NMEM), _full_spec(NMEM, NHID),
        ],
        out_specs=_row_spec(NHID),
        out_shape=jax.ShapeDtypeStruct((N, NHID), jnp.float32),
    )

    final = pl.pallas_call(
        _final_body,
        grid=(GRID,),
        in_specs=[
            _row_spec(NHID), _row_spec(NHID), _row_spec(NHID),
            _full_spec(NHID, NHID), _full_spec(NHID, NHID), _full_spec(1, NHID),
            _full_spec(NHID, NHID), _full_spec(1, NHID),
            _full_spec(NHID, NMEM), _full_spec(NMEM, NHID),
            _full_spec(NHID, NCLASS), _full_spec(NHID, NCLASS), _full_spec(1, NCLASS),
        ],
        out_specs=_row_spec(NCLASS),
        out_shape=jax.ShapeDtypeStruct((N, NCLASS), jnp.float32),
    )

    part0 = _seg_sum(h0, src, dst, zeros)
    h1 = layer(h0, part0[0], part0[1],
               W1_0[:NHID], W1_0[NHID:], b1_0.reshape(1, NHID),
               W2_0, b2_0.reshape(1, NHID), mem_0.T, mem_0)

    part1 = _seg_sum(h1, src, dst, zeros)
    out = final(h1, part1[0], part1[1],
                W1_1[:NHID], W1_1[NHID:], b1_1.reshape(1, NHID),
                W2_1, b2_1.reshape(1, NHID), mem_1.T, mem_1,
                W_lin[:NHID], W_lin[NHID:], b_lin.reshape(1, NCLASS))
    return out


# speedup vs baseline: 11.2698x; 1.1736x over previous
import functools

import jax
import jax.numpy as jnp
from jax import lax
from jax.experimental import pallas as pl
from jax.experimental.pallas import tpu as pltpu
from jax.experimental.pallas import tpu_sc as plsc

N = 10000
E = 320000
NFEAT = 128
NHID = 128
NCLASS = 40
NMEM = 8

NC = 2
NS = 16
NW = NC * NS
EW = E // NW
K = 80
CHUNKS = EW // K
NBUF = 2
RPT = 624
RPT_LAST = N - (NS - 1) * RPT

BLK = 2000
GRID = N // BLK


def _seg_sum_body(h_hbm, src_hbm, dst_hbm, zeros_hbm, out_hbm,
                  src_v, dst_v, rows0, rows1, acc, *gsems):
    rows = [rows0, rows1]
    cid = lax.axis_index("c")
    sid = lax.axis_index("s")
    wid = cid * NS + sid

    pltpu.sync_copy(src_hbm.at[wid], src_v)
    pltpu.sync_copy(dst_hbm.at[wid], dst_v)

    for b in range(NBUF):
        pltpu.async_copy(h_hbm.at[src_v.at[pl.ds(b * K, K)]], rows[b],
                         gsems[b])

    @pl.when(sid < NS - 1)
    def _():
        pltpu.sync_copy(zeros_hbm.at[pl.ds(sid * RPT, RPT)],
                        acc.at[pl.ds(sid * RPT, RPT)])

    @pl.when(sid == NS - 1)
    def _():
        pltpu.sync_copy(zeros_hbm.at[pl.ds((NS - 1) * RPT, RPT_LAST)],
                        acc.at[pl.ds((NS - 1) * RPT, RPT_LAST)])

    plsc.subcore_barrier()

    def step(jj, carry):
        for b in range(NBUF):
            j = jj * NBUF + b
            pltpu.make_async_copy(h_hbm.at[src_v.at[pl.ds(j * K, K)]],
                                  rows[b], gsems[b]).wait()
            pltpu.sync_copy(rows[b], acc.at[dst_v.at[j]], add=True)

            @pl.when(j + NBUF < CHUNKS)
            def _():
                pltpu.async_copy(
                    h_hbm.at[src_v.at[pl.ds((j + NBUF) * K, K)]], rows[b],
                    gsems[b])
        return carry

    lax.fori_loop(0, CHUNKS // NBUF, step, 0)
    for j in range((CHUNKS // NBUF) * NBUF, CHUNKS):
        b = j % NBUF
        pltpu.make_async_copy(h_hbm.at[src_v.at[pl.ds(j * K, K)]],
                              rows[b], gsems[b]).wait()
        pltpu.sync_copy(rows[b], acc.at[dst_v.at[j]], add=True)
    plsc.subcore_barrier()

    @pl.when(sid < NS - 1)
    def _():
        pltpu.sync_copy(acc.at[pl.ds(sid * RPT, RPT)],
                        out_hbm.at[cid, pl.ds(sid * RPT, RPT)])

    @pl.when(sid == NS - 1)
    def _():
        pltpu.sync_copy(acc.at[pl.ds((NS - 1) * RPT, RPT_LAST)],
                        out_hbm.at[cid, pl.ds((NS - 1) * RPT, RPT_LAST)])


_seg_sum = functools.partial(
    pl.kernel,
    _seg_sum_body,
    out_type=jax.ShapeDtypeStruct((NC, N, NHID), jnp.float32),
    mesh=plsc.VectorSubcoreMesh(core_axis_name="c", subcore_axis_name="s"),
    scratch_types=[
        pltpu.VMEM((EW,), jnp.int32),
        pltpu.VMEM((CHUNKS, K), jnp.int32),
        pltpu.VMEM((K, NHID), jnp.float32),
        pltpu.VMEM((K, NHID), jnp.float32),
        pltpu.VMEM_SHARED((N, NHID), jnp.float32),
    ] + [pltpu.SemaphoreType.DMA] * NBUF,
)()


def _pre_body(x_ref, w_ref, b_ref, o_ref):
    o_ref[...] = x_ref[...] @ w_ref[...] + b_ref[...]


def _layer_body(h_ref, pp_ref, w1x_ref, w1a_ref, b1_ref,
                w2_ref, b2_ref, memt_ref, mem_ref, o_ref):
    h = h_ref[...]
    agg = pp_ref[0] + pp_ref[1]
    t = h @ w1x_ref[...] + agg @ w1a_ref[...] + b1_ref[...]
    t = jnp.maximum(t, 0.0)
    u = t @ w2_ref[...] + b2_ref[...]
    u = jnp.maximum(u, 0.0)
    s = u @ memt_ref[...]
    s = s - jnp.max(s, axis=-1, keepdims=True)
    e = jnp.exp(s)
    p = e / jnp.sum(e, axis=-1, keepdims=True)
    o_ref[...] = p @ mem_ref[...]


def _final_body(h_ref, pp_ref, w1x_ref, w1a_ref, b1_ref,
                w2_ref, b2_ref, memt_ref, mem_ref,
                wla_ref, wlb_ref, bl_ref, o_ref):
    h = h_ref[...]
    agg = pp_ref[0] + pp_ref[1]
    t = h @ w1x_ref[...] + agg @ w1a_ref[...] + b1_ref[...]
    t = jnp.maximum(t, 0.0)
    u = t @ w2_ref[...] + b2_ref[...]
    u = jnp.maximum(u, 0.0)
    s = u @ memt_ref[...]
    s = s - jnp.max(s, axis=-1, keepdims=True)
    e = jnp.exp(s)
    p = e / jnp.sum(e, axis=-1, keepdims=True)
    h2 = p @ mem_ref[...]
    y = h @ wla_ref[...] + h2 @ wlb_ref[...] + bl_ref[...]
    m = jnp.max(y, axis=-1, keepdims=True)
    z = y - m
    o_ref[...] = z - jnp.log(jnp.sum(jnp.exp(z), axis=-1, keepdims=True))


def _row_spec(cols):
    return pl.BlockSpec((BLK, cols), lambda i: (i, 0))


def _part_spec():
    return pl.BlockSpec((NC, BLK, NHID), lambda i: (0, i, 0))


def _full_spec(rows, cols):
    return pl.BlockSpec((rows, cols), lambda i: (0, 0))


def kernel(x, edge_index, W_pre, b_pre, W1_0, b1_0, W2_0, b2_0, mem_0,
           W1_1, b1_1, W2_1, b2_1, mem_1, W_lin, b_lin):
    src = edge_index[0].astype(jnp.int32).reshape(NW, EW)
    dst = edge_index[1].astype(jnp.int32).reshape(NW, CHUNKS, K)
    zeros = jnp.zeros((N, NHID), jnp.float32)

    pre = pl.pallas_call(
        _pre_body,
        grid=(GRID,),
        in_specs=[_row_spec(NFEAT), _full_spec(NFEAT, NHID), _full_spec(1, NHID)],
        out_specs=_row_spec(NHID),
        out_shape=jax.ShapeDtypeStruct((N, NHID), jnp.float32),
    )
    h0 = pre(x, W_pre, b_pre.reshape(1, NHID))

    layer = pl.pallas_call(
        _layer_body,
        grid=(GRID,),
        in_specs=[
            _row_spec(NHID), _part_spec(),
            _full_spec(NHID, NHID), _full_spec(NHID, NHID), _full_spec(1, NHID),
            _full_spec(NHID, NHID), _full_spec(1, NHID),
            _full_spec(NHID, NMEM), _full_spec(NMEM, NHID),
        ],
        out_specs=_row_spec(NHID),
        out_shape=jax.ShapeDtypeStruct((N, NHID), jnp.float32),
    )

    final = pl.pallas_call(
        _final_body,
        grid=(GRID,),
        in_specs=[
            _row_spec(NHID), _part_spec(),
            _full_spec(NHID, NHID), _full_spec(NHID, NHID), _full_spec(1, NHID),
            _full_spec(NHID, NHID), _full_spec(1, NHID),
            _full_spec(NHID, NMEM), _full_spec(NMEM, NHID),
            _full_spec(NHID, NCLASS), _full_spec(NHID, NCLASS), _full_spec(1, NCLASS),
        ],
        out_specs=_row_spec(NCLASS),
        out_shape=jax.ShapeDtypeStruct((N, NCLASS), jnp.float32),
    )

    part0 = _seg_sum(h0, src, dst, zeros)
    h1 = layer(h0, part0,
               W1_0[:NHID], W1_0[NHID:], b1_0.reshape(1, NHID),
               W2_0, b2_0.reshape(1, NHID), mem_0.T, mem_0)

    part1 = _seg_sum(h1, src, dst, zeros)
    out = final(h1, part1,
                W1_1[:NHID], W1_1[NHID:], b1_1.reshape(1, NHID),
                W2_1, b2_1.reshape(1, NHID), mem_1.T, mem_1,
                W_lin[:NHID], W_lin[NHID:], b_lin.reshape(1, NCLASS))
    return out


# speedup vs baseline: 12.7927x; 1.1351x over previous
import functools

import jax
import jax.numpy as jnp
from jax import lax
from jax.experimental import pallas as pl
from jax.experimental.pallas import tpu as pltpu
from jax.experimental.pallas import tpu_sc as plsc

N = 10000
E = 320000
NFEAT = 128
NHID = 128
NCLASS = 40
NMEM = 8

NC = 2
NS = 16
NW = NC * NS
EW = E // NW
K = 80
CHUNKS = EW // K
NBUF = 3
RPT = 624
RPT_LAST = N - (NS - 1) * RPT
DPASS = 64

BLK = 2000
GRID = N // BLK


def _seg_sum_body(h_hbm, src_hbm, dst_hbm, zeros_hbm, out_hbm,
                  src_v, dst_v, rows0, rows1, rows2, acc, *gsems):
    rows = [rows0, rows1, rows2]
    cid = lax.axis_index("c")
    sid = lax.axis_index("s")
    wid = cid * NS + sid

    pltpu.sync_copy(src_hbm.at[wid], src_v)

    @pl.when(sid < NS - 1)
    def _():
        pltpu.sync_copy(zeros_hbm.at[pl.ds(sid * RPT, RPT)],
                        acc.at[pl.ds(sid * RPT, RPT)])

    @pl.when(sid == NS - 1)
    def _():
        pltpu.sync_copy(zeros_hbm.at[pl.ds((NS - 1) * RPT, RPT_LAST)],
                        acc.at[pl.ds((NS - 1) * RPT, RPT_LAST)])

    plsc.subcore_barrier()

    for lo, npass in ((0, DPASS), (DPASS, CHUNKS - DPASS)):
        pltpu.sync_copy(dst_hbm.at[wid, pl.ds(lo, npass)],
                        dst_v.at[pl.ds(0, npass)])
        for b in range(NBUF):
            pltpu.async_copy(h_hbm.at[src_v.at[pl.ds((lo + b) * K, K)]],
                             rows[b], gsems[b])

        def step(jj, carry):
            for b in range(NBUF):
                r = jj * NBUF + b
                j = lo + r
                pltpu.make_async_copy(h_hbm.at[src_v.at[pl.ds(j * K, K)]],
                                      rows[b], gsems[b]).wait()
                pltpu.sync_copy(rows[b], acc.at[dst_v.at[r]], add=True)

                @pl.when(j + NBUF < lo + npass)
                def _():
                    pltpu.async_copy(
                        h_hbm.at[src_v.at[pl.ds((j + NBUF) * K, K)]],
                        rows[b], gsems[b])
            return carry

        lax.fori_loop(0, npass // NBUF, step, 0)
        for r in range((npass // NBUF) * NBUF, npass):
            b = r % NBUF
            j = lo + r
            pltpu.make_async_copy(h_hbm.at[src_v.at[pl.ds(j * K, K)]],
                                  rows[b], gsems[b]).wait()
            pltpu.sync_copy(rows[b], acc.at[dst_v.at[r]], add=True)
    plsc.subcore_barrier()

    @pl.when(sid < NS - 1)
    def _():
        pltpu.sync_copy(acc.at[pl.ds(sid * RPT, RPT)],
                        out_hbm.at[cid, pl.ds(sid * RPT, RPT)])

    @pl.when(sid == NS - 1)
    def _():
        pltpu.sync_copy(acc.at[pl.ds((NS - 1) * RPT, RPT_LAST)],
                        out_hbm.at[cid, pl.ds((NS - 1) * RPT, RPT_LAST)])


_seg_sum = functools.partial(
    pl.kernel,
    _seg_sum_body,
    out_type=jax.ShapeDtypeStruct((NC, N, NHID), jnp.float32),
    mesh=plsc.VectorSubcoreMesh(core_axis_name="c", subcore_axis_name="s"),
    scratch_types=[
        pltpu.VMEM((EW,), jnp.int32),
        pltpu.VMEM((DPASS, K), jnp.int32),
        pltpu.VMEM((K, NHID), jnp.float32),
        pltpu.VMEM((K, NHID), jnp.float32),
        pltpu.VMEM((K, NHID), jnp.float32),
        pltpu.VMEM_SHARED((N, NHID), jnp.float32),
    ] + [pltpu.SemaphoreType.DMA] * NBUF,
)()


def _pre_body(x_ref, w_ref, b_ref, o_ref):
    o_ref[...] = x_ref[...] @ w_ref[...] + b_ref[...]


def _layer_body(h_ref, pp_ref, w1x_ref, w1a_ref, b1_ref,
                w2_ref, b2_ref, memt_ref, mem_ref, o_ref):
    h = h_ref[...]
    agg = pp_ref[0] + pp_ref[1]
    t = h @ w1x_ref[...] + agg @ w1a_ref[...] + b1_ref[...]
    t = jnp.maximum(t, 0.0)
    u = t @ w2_ref[...] + b2_ref[...]
    u = jnp.maximum(u, 0.0)
    s = u @ memt_ref[...]
    s = s - jnp.max(s, axis=-1, keepdims=True)
    e = jnp.exp(s)
    p = e / jnp.sum(e, axis=-1, keepdims=True)
    o_ref[...] = p @ mem_ref[...]


def _final_body(h_ref, pp_ref, w1x_ref, w1a_ref, b1_ref,
                w2_ref, b2_ref, memt_ref, mem_ref,
                wla_ref, wlb_ref, bl_ref, o_ref):
    h = h_ref[...]
    agg = pp_ref[0] + pp_ref[1]
    t = h @ w1x_ref[...] + agg @ w1a_ref[...] + b1_ref[...]
    t = jnp.maximum(t, 0.0)
    u = t @ w2_ref[...] + b2_ref[...]
    u = jnp.maximum(u, 0.0)
    s = u @ memt_ref[...]
    s = s - jnp.max(s, axis=-1, keepdims=True)
    e = jnp.exp(s)
    p = e / jnp.sum(e, axis=-1, keepdims=True)
    h2 = p @ mem_ref[...]
    y = h @ wla_ref[...] + h2 @ wlb_ref[...] + bl_ref[...]
    m = jnp.max(y, axis=-1, keepdims=True)
    z = y - m
    o_ref[...] = z - jnp.log(jnp.sum(jnp.exp(z), axis=-1, keepdims=True))


def _row_spec(cols):
    return pl.BlockSpec((BLK, cols), lambda i: (i, 0))


def _part_spec():
    return pl.BlockSpec((NC, BLK, NHID), lambda i: (0, i, 0))


def _full_spec(rows, cols):
    return pl.BlockSpec((rows, cols), lambda i: (0, 0))


def kernel(x, edge_index, W_pre, b_pre, W1_0, b1_0, W2_0, b2_0, mem_0,
           W1_1, b1_1, W2_1, b2_1, mem_1, W_lin, b_lin):
    src = edge_index[0].astype(jnp.int32).reshape(NW, EW)
    dst = edge_index[1].astype(jnp.int32).reshape(NW, CHUNKS, K)
    zeros = jnp.zeros((N, NHID), jnp.float32)

    pre = pl.pallas_call(
        _pre_body,
        grid=(GRID,),
        in_specs=[_row_spec(NFEAT), _full_spec(NFEAT, NHID), _full_spec(1, NHID)],
        out_specs=_row_spec(NHID),
        out_shape=jax.ShapeDtypeStruct((N, NHID), jnp.float32),
    )
    h0 = pre(x, W_pre, b_pre.reshape(1, NHID))

    layer = pl.pallas_call(
        _layer_body,
        grid=(GRID,),
        in_specs=[
            _row_spec(NHID), _part_spec(),
            _full_spec(NHID, NHID), _full_spec(NHID, NHID), _full_spec(1, NHID),
            _full_spec(NHID, NHID), _full_spec(1, NHID),
            _full_spec(NHID, NMEM), _full_spec(NMEM, NHID),
        ],
        out_specs=_row_spec(NHID),
        out_shape=jax.ShapeDtypeStruct((N, NHID), jnp.float32),
    )

    final = pl.pallas_call(
        _final_body,
        grid=(GRID,),
        in_specs=[
            _row_spec(NHID), _part_spec(),
            _full_spec(NHID, NHID), _full_spec(NHID, NHID), _full_spec(1, NHID),
            _full_spec(NHID, NHID), _full_spec(1, NHID),
            _full_spec(NHID, NMEM), _full_spec(NMEM, NHID),
            _full_spec(NHID, NCLASS), _full_spec(NHID, NCLASS), _full_spec(1, NCLASS),
        ],
        out_specs=_row_spec(NCLASS),
        out_shape=jax.ShapeDtypeStruct((N, NCLASS), jnp.float32),
    )

    part0 = _seg_sum(h0, src, dst, zeros)
    h1 = layer(h0, part0,
               W1_0[:NHID], W1_0[NHID:], b1_0.reshape(1, NHID),
               W2_0, b2_0.reshape(1, NHID), mem_0.T, mem_0)

    part1 = _seg_sum(h1, src, dst, zeros)
    out = final(h1, part1,
                W1_1[:NHID], W1_1[NHID:], b1_1.reshape(1, NHID),
                W2_1, b2_1.reshape(1, NHID), mem_1.T, mem_1,
                W_lin[:NHID], W_lin[NHID:], b_lin.reshape(1, NCLASS))
    return out
